# baseline (speedup 1.0000x reference)
import functools

import jax
import jax.numpy as jnp
from jax.experimental import pallas as pl
from jax.experimental.pallas import tpu as pltpu

_ID_DIM = 32
_LLM_DIM = 768
_D = _ID_DIM + _LLM_DIM
_OUT_DIM = 32
_E = 8
_H = 2 * _OUT_DIM
_EH = _E * _H
_GH = 2 * _E
_NW = _EH + _GH
_B = 16384
_BT = 1024

_R_WS = 0
_R_W2 = _D
_R_BIAS = _R_W2 + _EH
_R_B2 = _R_BIAS + 8
_R_WG2T = _R_B2 + 8
_R_BG2 = _R_WG2T + 8
_ROWS = _R_BG2 + 8


def _fused_body(id_ref, llm_ref, wp_ref, out_ref):
    f32 = jnp.float32
    bf16 = jnp.bfloat16
    idb = id_ref[...].astype(bf16)
    llm = llm_ref[...].astype(bf16)

    hall = jnp.maximum(
        jnp.dot(idb, wp_ref[:_ID_DIM, :].astype(bf16),
                preferred_element_type=f32)
        + jnp.dot(llm, wp_ref[_ID_DIM:_D, :].astype(bf16),
                  preferred_element_type=f32)
        + wp_ref[_R_BIAS:_R_BIAS + 1, :], 0.0)
    h = hall[:, :_EH]
    ght = hall[:, _EH:].T

    logits = (jnp.dot(wp_ref[_R_WG2T:_R_WG2T + _E, :_GH], ght,
                      preferred_element_type=f32)
              + wp_ref[_R_BG2:_R_BG2 + _E, :1])

    sub = jax.lax.broadcasted_iota(jnp.int32, logits.shape, 0)
    m1 = jnp.max(logits, axis=0, keepdims=True)
    i1 = jnp.min(jnp.where(logits == m1, sub, _E), axis=0, keepdims=True)
    oh1 = sub == i1
    masked = jnp.where(oh1, -jnp.inf, logits)
    m2 = jnp.max(masked, axis=0, keepdims=True)
    i2 = jnp.min(jnp.where(masked == m2, sub, _E), axis=0, keepdims=True)
    oh2 = sub == i2
    wtop = 1.0 / (1.0 + jnp.exp(m2 - m1))
    wvec = (jnp.where(oh1, wtop, 0.0) + jnp.where(oh2, 1.0 - wtop, 0.0)).T

    lane = jax.lax.broadcasted_iota(jnp.int32, (_E, _EH), 1) // _H
    erow = jax.lax.broadcasted_iota(jnp.int32, (_E, _EH), 0)
    exp_mat = (lane == erow).astype(f32)
    wexp = jnp.dot(wvec, exp_mat, preferred_element_type=f32)
    out = jnp.dot((h * wexp).astype(bf16),
                  wp_ref[_R_W2:_R_W2 + _EH, :_OUT_DIM].astype(bf16),
                  preferred_element_type=f32)
    out_ref[...] = out + jnp.dot(wvec, wp_ref[_R_B2:_R_B2 + _E, :_OUT_DIM],
                                 preferred_element_type=f32)


@functools.partial(jax.jit, static_argnames=())
def kernel(id_emb, llm_emb, W1, b1, W2, b2, Wg1, bg1, Wg2, bg2):
    padw = lambda a: jnp.pad(a, ((0, 0), (0, _NW - a.shape[1])))
    rows8 = lambda a: jnp.pad(a, ((0, (-a.shape[0]) % 8), (0, 0)))
    ws = jnp.concatenate([jnp.transpose(W1, (1, 0, 2)).reshape(_D, _EH),
                          Wg1], axis=1)
    wpack = jnp.concatenate([
        ws,
        padw(W2.reshape(_EH, _OUT_DIM)),
        rows8(padw(jnp.concatenate([b1.reshape(1, _EH),
                                    bg1.reshape(1, _GH)], axis=1))),
        rows8(padw(b2)),
        rows8(padw(Wg2.T)),
        rows8(padw(bg2.reshape(_E, 1))),
    ], axis=0)

    grid = (_B // _BT,)
    return pl.pallas_call(
        _fused_body,
        grid=grid,
        in_specs=[
            pl.BlockSpec((_BT, _ID_DIM), lambda i: (i, 0)),
            pl.BlockSpec((_BT, _LLM_DIM), lambda i: (i, 0)),
            pl.BlockSpec((_ROWS, _NW), lambda i: (0, 0)),
        ],
        out_specs=pl.BlockSpec((_BT, _OUT_DIM), lambda i: (i, 0)),
        out_shape=jax.ShapeDtypeStruct((_B, _OUT_DIM), jnp.float32),
        compiler_params=pltpu.CompilerParams(
            vmem_limit_bytes=120 * 1024 * 1024),
    )(id_emb, llm_emb, wpack)


# speedup vs baseline: 1.0186x; 1.0186x over previous
import functools

import jax
import jax.numpy as jnp
from jax.experimental import pallas as pl
from jax.experimental.pallas import tpu as pltpu

_ID_DIM = 32
_LLM_DIM = 768
_D = _ID_DIM + _LLM_DIM
_OUT_DIM = 32
_E = 8
_H = 2 * _OUT_DIM
_EH = _E * _H
_GH = 2 * _E
_NW = _EH + _GH
_B = 16384
_BT = 2048

_R_WS = 0
_R_W2 = _D
_R_BIAS = _R_W2 + _EH
_R_B2 = _R_BIAS + 8
_R_WG2T = _R_B2 + 8
_R_BG2 = _R_WG2T + 8
_ROWS = _R_BG2 + 8


def _fused_body(id_ref, llm_ref, wp_ref, out_ref):
    f32 = jnp.float32
    bf16 = jnp.bfloat16
    idb = id_ref[...].astype(bf16)
    llm = llm_ref[...].astype(bf16)

    hall = jnp.maximum(
        jnp.dot(idb, wp_ref[:_ID_DIM, :].astype(bf16),
                preferred_element_type=f32)
        + jnp.dot(llm, wp_ref[_ID_DIM:_D, :].astype(bf16),
                  preferred_element_type=f32)
        + wp_ref[_R_BIAS:_R_BIAS + 1, :], 0.0)
    h = hall[:, :_EH]

    logits = jax.lax.dot_general(
        wp_ref[_R_WG2T:_R_WG2T + _E, :_GH], hall[:, _EH:],
        (((1,), (1,)), ((), ())),
        preferred_element_type=f32) + wp_ref[_R_BG2:_R_BG2 + _E, :1]

    sub = jax.lax.broadcasted_iota(jnp.int32, logits.shape, 0)
    m1 = jnp.max(logits, axis=0, keepdims=True)
    i1 = jnp.min(jnp.where(logits == m1, sub, _E), axis=0, keepdims=True)
    oh1 = sub == i1
    masked = jnp.where(oh1, -jnp.inf, logits)
    m2 = jnp.max(masked, axis=0, keepdims=True)
    i2 = jnp.min(jnp.where(masked == m2, sub, _E), axis=0, keepdims=True)
    oh2 = sub == i2
    wtop = 1.0 / (1.0 + jnp.exp(m2 - m1))
    wvt = jnp.where(oh1, wtop, 0.0) + jnp.where(oh2, 1.0 - wtop, 0.0)

    lane = jax.lax.broadcasted_iota(jnp.int32, (_E, _EH), 1) // _H
    erow = jax.lax.broadcasted_iota(jnp.int32, (_E, _EH), 0)
    exp_mat = (lane == erow).astype(f32)
    wexp = jax.lax.dot_general(wvt, exp_mat, (((0,), (0,)), ((), ())),
                               preferred_element_type=f32)
    out = jnp.dot((h * wexp).astype(bf16),
                  wp_ref[_R_W2:_R_W2 + _EH, :_OUT_DIM].astype(bf16),
                  preferred_element_type=f32)
    out_ref[...] = out + jax.lax.dot_general(
        wvt, wp_ref[_R_B2:_R_B2 + _E, :_OUT_DIM], (((0,), (0,)), ((), ())),
        preferred_element_type=f32)


@functools.partial(jax.jit, static_argnames=())
def kernel(id_emb, llm_emb, W1, b1, W2, b2, Wg1, bg1, Wg2, bg2):
    padw = lambda a: jnp.pad(a, ((0, 0), (0, _NW - a.shape[1])))
    rows8 = lambda a: jnp.pad(a, ((0, (-a.shape[0]) % 8), (0, 0)))
    ws = jnp.concatenate([jnp.transpose(W1, (1, 0, 2)).reshape(_D, _EH),
                          Wg1], axis=1)
    wpack = jnp.concatenate([
        ws,
        padw(W2.reshape(_EH, _OUT_DIM)),
        rows8(padw(jnp.concatenate([b1.reshape(1, _EH),
                                    bg1.reshape(1, _GH)], axis=1))),
        rows8(padw(b2)),
        rows8(padw(Wg2.T)),
        rows8(padw(bg2.reshape(_E, 1))),
    ], axis=0)

    grid = (_B // _BT,)
    return pl.pallas_call(
        _fused_body,
        grid=grid,
        in_specs=[
            pl.BlockSpec((_BT, _ID_DIM), lambda i: (i, 0)),
            pl.BlockSpec((_BT, _LLM_DIM), lambda i: (i, 0)),
            pl.BlockSpec((_ROWS, _NW), lambda i: (0, 0)),
        ],
        out_specs=pl.BlockSpec((_BT, _OUT_DIM), lambda i: (i, 0)),
        out_shape=jax.ShapeDtypeStruct((_B, _OUT_DIM), jnp.float32),
        compiler_params=pltpu.CompilerParams(
            vmem_limit_bytes=120 * 1024 * 1024),
    )(id_emb, llm_emb, wpack)


# speedup vs baseline: 1.0470x; 1.0279x over previous
import functools

import jax
import jax.numpy as jnp
from jax.experimental import pallas as pl
from jax.experimental.pallas import tpu as pltpu

_ID_DIM = 32
_LLM_DIM = 768
_D = _ID_DIM + _LLM_DIM
_OUT_DIM = 32
_E = 8
_H = 2 * _OUT_DIM
_EH = _E * _H
_GH = 2 * _E
_NW = _EH + _GH
_B = 16384
_BT = 2048

_R_WS = 0
_R_W2 = _D
_R_BIAS = _R_W2 + _EH
_R_B2 = _R_BIAS + 8
_R_WG2T = _R_B2 + 8
_R_BG2 = _R_WG2T + 8
_ROWS = _R_BG2 + 8


def _fused_body(id_ref, llm_ref, wp_ref, out_ref):
    f32 = jnp.float32
    bf16 = jnp.bfloat16
    idb = id_ref[...].astype(bf16)
    llm = llm_ref[...].astype(bf16)

    hall = jnp.maximum(
        jnp.dot(idb, wp_ref[:_ID_DIM, :].astype(bf16),
                preferred_element_type=f32)
        + jnp.dot(llm, wp_ref[_ID_DIM:_D, :].astype(bf16),
                  preferred_element_type=f32)
        + wp_ref[_R_BIAS:_R_BIAS + 1, :], 0.0).astype(bf16)
    h = hall[:, :_EH]

    logits = jax.lax.dot_general(
        wp_ref[_R_WG2T:_R_WG2T + _E, :_GH].astype(bf16), hall[:, _EH:],
        (((1,), (1,)), ((), ())),
        preferred_element_type=f32) + wp_ref[_R_BG2:_R_BG2 + _E, :1]

    sub = jax.lax.broadcasted_iota(jnp.int32, logits.shape, 0)
    m1 = jnp.max(logits, axis=0, keepdims=True)
    i1 = jnp.min(jnp.where(logits == m1, sub, _E), axis=0, keepdims=True)
    oh1 = sub == i1
    masked = jnp.where(oh1, -jnp.inf, logits)
    m2 = jnp.max(masked, axis=0, keepdims=True)
    i2 = jnp.min(jnp.where(masked == m2, sub, _E), axis=0, keepdims=True)
    oh2 = sub == i2
    wtop = 1.0 / (1.0 + jnp.exp(m2 - m1))
    wvt = jnp.where(oh1, wtop, 0.0) + jnp.where(oh2, 1.0 - wtop, 0.0)

    lane = jax.lax.broadcasted_iota(jnp.int32, (_E, _EH), 1) // _H
    erow = jax.lax.broadcasted_iota(jnp.int32, (_E, _EH), 0)
    exp_mat = (lane == erow).astype(bf16)
    wexp = jax.lax.dot_general(wvt.astype(bf16), exp_mat,
                               (((0,), (0,)), ((), ())),
                               preferred_element_type=f32).astype(bf16)
    out = jnp.dot(h * wexp,
                  wp_ref[_R_W2:_R_W2 + _EH, :_OUT_DIM].astype(bf16),
                  preferred_element_type=f32)
    out_ref[...] = out + jax.lax.dot_general(
        wvt, wp_ref[_R_B2:_R_B2 + _E, :_OUT_DIM], (((0,), (0,)), ((), ())),
        preferred_element_type=f32)


@functools.partial(jax.jit, static_argnames=())
def kernel(id_emb, llm_emb, W1, b1, W2, b2, Wg1, bg1, Wg2, bg2):
    padw = lambda a: jnp.pad(a, ((0, 0), (0, _NW - a.shape[1])))
    rows8 = lambda a: jnp.pad(a, ((0, (-a.shape[0]) % 8), (0, 0)))
    ws = jnp.concatenate([jnp.transpose(W1, (1, 0, 2)).reshape(_D, _EH),
                          Wg1], axis=1)
    wpack = jnp.concatenate([
        ws,
        padw(W2.reshape(_EH, _OUT_DIM)),
        rows8(padw(jnp.concatenate([b1.reshape(1, _EH),
                                    bg1.reshape(1, _GH)], axis=1))),
        rows8(padw(b2)),
        rows8(padw(Wg2.T)),
        rows8(padw(bg2.reshape(_E, 1))),
    ], axis=0)

    grid = (_B // _BT,)
    return pl.pallas_call(
        _fused_body,
        grid=grid,
        in_specs=[
            pl.BlockSpec((_BT, _ID_DIM), lambda i: (i, 0)),
            pl.BlockSpec((_BT, _LLM_DIM), lambda i: (i, 0)),
            pl.BlockSpec((_ROWS, _NW), lambda i: (0, 0)),
        ],
        out_specs=pl.BlockSpec((_BT, _OUT_DIM), lambda i: (i, 0)),
        out_shape=jax.ShapeDtypeStruct((_B, _OUT_DIM), jnp.float32),
        compiler_params=pltpu.CompilerParams(
            vmem_limit_bytes=120 * 1024 * 1024),
    )(id_emb, llm_emb, wpack)


# speedup vs baseline: 1.0615x; 1.0138x over previous
import functools

import jax
import jax.numpy as jnp
from jax.experimental import pallas as pl
from jax.experimental.pallas import tpu as pltpu

_ID_DIM = 32
_LLM_DIM = 768
_D = _ID_DIM + _LLM_DIM
_OUT_DIM = 32
_E = 8
_H = 2 * _OUT_DIM
_EH = _E * _H
_GH = 2 * _E
_NW = _EH + _GH
_B = 16384
_BT = 2048

_R_WS = 0
_R_W2 = _D
_R_BIAS = _R_W2 + _EH
_R_B2 = _R_BIAS + 8
_R_WG2T = _R_B2 + 8
_R_BG2 = _R_WG2T + 8
_ROWS = _R_BG2 + 8


def _fused_body(id_ref, llm_ref, wp_ref, out_ref):
    f32 = jnp.float32
    bf16 = jnp.bfloat16
    idb = id_ref[...].astype(bf16)
    llm = llm_ref[...].astype(bf16)

    hall = jnp.maximum(
        jnp.dot(idb, wp_ref[:_ID_DIM, :].astype(bf16),
                preferred_element_type=f32)
        + jnp.dot(llm, wp_ref[_ID_DIM:_D, :].astype(bf16),
                  preferred_element_type=f32)
        + wp_ref[_R_BIAS:_R_BIAS + 1, :], 0.0).astype(bf16)
    h = hall[:, :_EH]

    logits = jax.lax.dot_general(
        wp_ref[_R_WG2T:_R_WG2T + _E, :_GH].astype(bf16), hall[:, _EH:],
        (((1,), (1,)), ((), ())),
        preferred_element_type=f32) + wp_ref[_R_BG2:_R_BG2 + _E, :1]

    sub = jax.lax.broadcasted_iota(jnp.int32, logits.shape, 0)
    m1 = jnp.max(logits, axis=0, keepdims=True)
    i1 = jnp.min(jnp.where(logits == m1, sub, _E), axis=0, keepdims=True)
    oh1 = sub == i1
    masked = jnp.where(oh1, -jnp.inf, logits)
    m2 = jnp.max(masked, axis=0, keepdims=True)
    i2 = jnp.min(jnp.where(masked == m2, sub, _E), axis=0, keepdims=True)
    oh2 = sub == i2
    wtop = 1.0 / (1.0 + jnp.exp(m2 - m1))
    wvt = jnp.where(oh1, wtop, 0.0) + jnp.where(oh2, 1.0 - wtop, 0.0)

    lane = jax.lax.broadcasted_iota(jnp.int32, (_E, _EH), 1) // _H
    erow = jax.lax.broadcasted_iota(jnp.int32, (_E, _EH), 0)
    exp_mat = (lane == erow).astype(bf16)
    wexp = jax.lax.dot_general(wvt.astype(bf16), exp_mat,
                               (((0,), (0,)), ((), ())),
                               preferred_element_type=f32).astype(bf16)
    out = jnp.dot(h * wexp,
                  wp_ref[_R_W2:_R_W2 + _EH, :_OUT_DIM].astype(bf16),
                  preferred_element_type=f32)
    out_ref[...] = out + jax.lax.dot_general(
        wvt, wp_ref[_R_B2:_R_B2 + _E, :_OUT_DIM], (((0,), (0,)), ((), ())),
        preferred_element_type=f32)


@functools.partial(jax.jit, static_argnames=())
def kernel(id_emb, llm_emb, W1, b1, W2, b2, Wg1, bg1, Wg2, bg2):
    padw = lambda a: jnp.pad(a, ((0, 0), (0, _NW - a.shape[1])))
    rows8 = lambda a: jnp.pad(a, ((0, (-a.shape[0]) % 8), (0, 0)))
    ws = jnp.concatenate([jnp.transpose(W1, (1, 0, 2)).reshape(_D, _EH),
                          Wg1], axis=1)
    wpack = jnp.concatenate([
        ws,
        padw(W2.reshape(_EH, _OUT_DIM)),
        rows8(padw(jnp.concatenate([b1.reshape(1, _EH),
                                    bg1.reshape(1, _GH)], axis=1))),
        rows8(padw(b2)),
        rows8(padw(Wg2.T)),
        rows8(padw(bg2.reshape(_E, 1))),
    ], axis=0)

    grid = (_B // _BT,)
    return pl.pallas_call(
        _fused_body,
        grid=grid,
        in_specs=[
            pl.BlockSpec((_BT, _ID_DIM), lambda i: (i, 0)),
            pl.BlockSpec((_BT, _LLM_DIM), lambda i: (i, 0)),
            pl.BlockSpec((_ROWS, _NW), lambda i: (0, 0)),
        ],
        out_specs=pl.BlockSpec((_BT, _OUT_DIM), lambda i: (i, 0)),
        out_shape=jax.ShapeDtypeStruct((_B, _OUT_DIM), jnp.float32),
        compiler_params=pltpu.CompilerParams(
            dimension_semantics=("parallel",),
            vmem_limit_bytes=120 * 1024 * 1024),
    )(id_emb, llm_emb, wpack)
